# Initial kernel scaffold; baseline (speedup 1.0000x reference)
import jax
import jax.numpy as jnp
from jax.experimental import pallas as pl


def kernel(feature_map, rois, img_size):
    raise NotImplementedError("write your pallas kernel here")


# speedup vs baseline: 9.1206x; 9.1206x over previous
import jax
import jax.numpy as jnp
from jax.experimental import pallas as pl
from jax.experimental.pallas import tpu as pltpu

_POOL = 7
_CROP = 2 * _POOL
_BN = 8
_WPAD = 88


def _roi_pool_kernel(params_ref, img_ref, out_ref, yscr):
    b = pl.program_id(0)
    H = img_ref.shape[0]
    W = img_ref.shape[1]
    C = img_ref.shape[2]

    yscr[:, :, W:, :] = jnp.zeros((_POOL, 2, _WPAD - W, C), jnp.float32)

    sub16 = jax.lax.broadcasted_iota(jnp.int32, (16, C), 0)

    def do_roi(r, carry):
        base = (b * _BN + r) * 4
        ys0 = params_ref[base + 0]
        ysp = params_ref[base + 1]
        xs0 = params_ref[base + 2]
        xsp = params_ref[base + 3]

        def yrow(i, carry2):
            t = ys0 + i.astype(jnp.float32) * ysp
            y0 = jnp.clip(jnp.floor(t).astype(jnp.int32), 0, H - 2)
            w = t - y0.astype(jnp.float32)
            rows = img_ref[pl.ds(y0, 2)]
            lerped = rows[0] + w * (rows[1] - rows[0])
            yscr[pl.ds(i // 2, 1), pl.ds(i % 2, 1), :W, :] = lerped[None, None]
            return carry2

        jax.lax.fori_loop(0, _CROP, yrow, 0)

        for px in range(_POOL):
            ta = xs0 + jnp.float32(2 * px) * xsp
            tb = xs0 + jnp.float32(2 * px + 1) * xsp
            x0a = jnp.clip(jnp.floor(ta).astype(jnp.int32), 0, W - 2)
            x0b = jnp.clip(jnp.floor(tb).astype(jnp.int32), 0, W - 2)
            wa = ta - x0a.astype(jnp.float32)
            wb = tb - x0b.astype(jnp.float32)
            al = pl.multiple_of((x0a >> 3) << 3, 8)
            offa = x0a - al
            offb = x0b - al
            win = yscr[:, :, pl.ds(al, 16), :]
            wva = (jnp.where(sub16 == offa, 1.0 - wa, 0.0)
                   + jnp.where(sub16 == offa + 1, wa, 0.0))
            wvb = (jnp.where(sub16 == offb, 1.0 - wb, 0.0)
                   + jnp.where(sub16 == offb + 1, wb, 0.0))
            va = jnp.sum(win * wva[None, None], axis=2)
            vb = jnp.sum(win * wvb[None, None], axis=2)
            m = jnp.maximum(va, vb)
            mp = jnp.max(m, axis=1)
            out_ref[pl.ds(r, 1), :, px, :] = mp[None]
        return carry

    jax.lax.fori_loop(0, _BN, do_roi, 0)


def kernel(feature_map, rois, img_size):
    _, H, W, C = feature_map.shape
    N = rois.shape[0]
    img = feature_map[0]

    img_h = img_size[0].astype(jnp.float32) - 1.0
    img_w = img_size[1].astype(jnp.float32) - 1.0
    fh = jnp.float32(H - 1)
    fw = jnp.float32(W - 1)
    y1 = rois[:, 1] / img_h
    x1 = rois[:, 0] / img_w
    y2 = rois[:, 3] / img_h
    x2 = rois[:, 2] / img_w
    ystart = y1 * fh
    ystep = (y2 - y1) * fh / (_CROP - 1)
    xstart = x1 * fw
    xstep = (x2 - x1) * fw / (_CROP - 1)
    params = jnp.stack([ystart, ystep, xstart, xstep], axis=1).reshape(-1)

    grid = (N // _BN,)

    return pl.pallas_call(
        _roi_pool_kernel,
        out_shape=jax.ShapeDtypeStruct((N, _POOL, _POOL, C), jnp.float32),
        grid_spec=pltpu.PrefetchScalarGridSpec(
            num_scalar_prefetch=1,
            grid=grid,
            in_specs=[
                pl.BlockSpec((H, W, C), lambda b, p: (0, 0, 0)),
            ],
            out_specs=pl.BlockSpec((_BN, _POOL, _POOL, C), lambda b, p: (b, 0, 0, 0)),
            scratch_shapes=[
                pltpu.VMEM((_POOL, 2, _WPAD, C), jnp.float32),
            ],
        ),
        compiler_params=pltpu.CompilerParams(
            dimension_semantics=("parallel",),
            vmem_limit_bytes=48 * 1024 * 1024,
        ),
        name="roi_pool",
    )(params, img)


# speedup vs baseline: 26.4327x; 2.8982x over previous
import jax
import jax.numpy as jnp
from jax.experimental import pallas as pl
from jax.experimental.pallas import tpu as pltpu

_POOL = 7
_CROP = 2 * _POOL
_BN = 8


def _make_kernel(H, W, C):
    WC = W * C

    def _roi_pool_kernel(params_ref, img_ref, out_ref, wscr):
        b = pl.program_id(0)

        hi = jax.lax.broadcasted_iota(jnp.int32, (8, H), 1).astype(jnp.float32)
        si = jax.lax.broadcasted_iota(jnp.int32, (8, H), 0).astype(jnp.float32)

        def build8(ys0, ysp, par):
            t = ys0 + (2.0 * si + par) * ysp
            y0 = jnp.clip(jnp.floor(t), 0.0, H - 2)
            w = t - y0
            return (jnp.where(hi == y0, 1.0 - w, 0.0)
                    + jnp.where(hi == y0 + 1.0, w, 0.0))

        mats = []
        for par in (0.0, 1.0):
            for r in range(_BN):
                base = (b * _BN + r) * 4
                ys0 = params_ref[base + 0]
                ysp = params_ref[base + 1]
                mats.append(build8(ys0, ysp, par))
        wy = jnp.concatenate(mats, axis=0).astype(jnp.bfloat16)

        yall = jax.lax.dot_general(
            wy, img_ref[...], (((1,), (0,)), ((), ())),
            preferred_element_type=jnp.float32)
        wscr[...] = yall.reshape(2, _BN, 8, WC)

        def do_roi(r, carry):
            base = (b * _BN + r) * 4
            xs0 = params_ref[base + 2]
            xsp = params_ref[base + 3]
            for px in range(_POOL):
                ta = xs0 + jnp.float32(2 * px) * xsp
                tb = xs0 + jnp.float32(2 * px + 1) * xsp
                x0a = jnp.clip(jnp.floor(ta).astype(jnp.int32), 0, W - 2)
                x0b = jnp.clip(jnp.floor(tb).astype(jnp.int32), 0, W - 2)
                wa = ta - x0a.astype(jnp.float32)
                wb = tb - x0b.astype(jnp.float32)
                offa = pl.multiple_of(x0a * C, C)
                offb = pl.multiple_of(x0b * C, C)

                def taps(par, off, w):
                    c0 = wscr[par, pl.ds(r, 1), :_POOL, pl.ds(off, C)]
                    c1 = wscr[par, pl.ds(r, 1), :_POOL, pl.ds(off + C, C)]
                    return c0 + w * (c1 - c0)

                m = jnp.maximum(
                    jnp.maximum(taps(0, offa, wa), taps(0, offb, wb)),
                    jnp.maximum(taps(1, offa, wa), taps(1, offb, wb)))
                out_ref[pl.ds(r, 1), :, px, :] = m
            return carry

        jax.lax.fori_loop(0, _BN, do_roi, 0)

    return _roi_pool_kernel


def kernel(feature_map, rois, img_size):
    _, H, W, C = feature_map.shape
    N = rois.shape[0]
    img = feature_map[0].reshape(H, W * C).astype(jnp.bfloat16)

    img_h = img_size[0].astype(jnp.float32) - 1.0
    img_w = img_size[1].astype(jnp.float32) - 1.0
    fh = jnp.float32(H - 1)
    fw = jnp.float32(W - 1)
    y1 = rois[:, 1] / img_h
    x1 = rois[:, 0] / img_w
    y2 = rois[:, 3] / img_h
    x2 = rois[:, 2] / img_w
    ystart = y1 * fh
    ystep = (y2 - y1) * fh / (_CROP - 1)
    xstart = x1 * fw
    xstep = (x2 - x1) * fw / (_CROP - 1)
    params = jnp.stack([ystart, ystep, xstart, xstep], axis=1).reshape(-1)

    grid = (N // _BN,)

    return pl.pallas_call(
        _make_kernel(H, W, C),
        out_shape=jax.ShapeDtypeStruct((N, _POOL, _POOL, C), jnp.float32),
        grid_spec=pltpu.PrefetchScalarGridSpec(
            num_scalar_prefetch=1,
            grid=grid,
            in_specs=[
                pl.BlockSpec((H, W * C), lambda b, p: (0, 0)),
            ],
            out_specs=pl.BlockSpec((_BN, _POOL, _POOL, C), lambda b, p: (b, 0, 0, 0)),
            scratch_shapes=[
                pltpu.VMEM((2, _BN, 8, W * C), jnp.float32),
            ],
        ),
        compiler_params=pltpu.CompilerParams(
            dimension_semantics=("arbitrary",),
            vmem_limit_bytes=48 * 1024 * 1024,
        ),
        name="roi_pool",
    )(params, img)
